# baseline (speedup 1.0000x reference)
import jax
import jax.numpy as jnp
from jax.experimental import pallas as pl
from jax.experimental.pallas import tpu as pltpu


def _dma_body(x_ref, pe_ref, out_ref, sem_x, sem_pe):
    B, L, D = x_ref.shape
    P = pe_ref.shape[1]
    xcopies = []
    pcopies = []
    for b in range(B):
        c = pltpu.make_async_copy(x_ref.at[b], out_ref.at[b, :, :D], sem_x)
        c.start()
        xcopies.append(c)
        c = pltpu.make_async_copy(pe_ref, out_ref.at[b, :, D:], sem_pe)
        c.start()
        pcopies.append(c)
    for c in xcopies:
        c.wait()
    for c in pcopies:
        c.wait()


def kernel(x, pe_weight, pos):
    B, L, D = x.shape
    P = pe_weight.shape[1]
    del pos
    return pl.pallas_call(
        _dma_body,
        in_specs=[
            pl.BlockSpec(memory_space=pl.ANY),
            pl.BlockSpec(memory_space=pl.ANY),
        ],
        out_specs=pl.BlockSpec(memory_space=pl.ANY),
        out_shape=jax.ShapeDtypeStruct((B, L, D + P), x.dtype),
        scratch_shapes=[pltpu.SemaphoreType.DMA, pltpu.SemaphoreType.DMA],
    )(x, pe_weight)


# speedup vs baseline: 34.2691x; 34.2691x over previous
import functools

import jax
import jax.numpy as jnp
from jax import lax
from jax.experimental import pallas as pl
from jax.experimental.pallas import tpu as pltpu
from jax.experimental.pallas import tpu_sc as plsc

_BLK = 2048


def _xcopy_body(x_ref, out_ref):
    out_ref[0, :, : x_ref.shape[2]] = x_ref[0]


def _pe_body(_, xpos_ref, out_ref):
    out_ref[0] = xpos_ref[...]


def _sc_gather(pe_weight, pos):
    V, P = pe_weight.shape
    L = pos.shape[0]
    info = plsc.get_sparse_core_info()
    nw = info.num_cores * info.num_subcores
    rows_per_w = L // nw
    mesh = plsc.VectorSubcoreMesh(core_axis_name="c", subcore_axis_name="s")

    @functools.partial(
        pl.kernel,
        mesh=mesh,
        out_type=jax.ShapeDtypeStruct((L, P), pe_weight.dtype),
        scratch_types=[
            pltpu.VMEM((rows_per_w,), jnp.int32),
            pltpu.VMEM((rows_per_w, P), pe_weight.dtype),
            pltpu.SemaphoreType.DMA,
        ],
    )
    def gather_k(pe_hbm, pos_hbm, out_hbm, idx_v, rows_v, sem):
        wid = lax.axis_index("s") * info.num_cores + lax.axis_index("c")
        base = wid * rows_per_w
        pltpu.sync_copy(pos_hbm.at[pl.ds(base, rows_per_w)], idx_v)
        pltpu.async_copy(pe_hbm.at[idx_v], rows_v, sem).wait()
        pltpu.sync_copy(rows_v, out_hbm.at[pl.ds(base, rows_per_w)])

    return gather_k(pe_weight, pos)


def kernel(x, pe_weight, pos):
    B, L, D = x.shape
    P = pe_weight.shape[1]
    W = D + P

    x_pos = _sc_gather(pe_weight, pos)

    out1 = pl.pallas_call(
        _xcopy_body,
        grid=(L // _BLK, B),
        in_specs=[pl.BlockSpec((1, _BLK, D), lambda i, b: (b, i, 0))],
        out_specs=pl.BlockSpec((1, _BLK, W), lambda i, b: (b, i, 0)),
        out_shape=jax.ShapeDtypeStruct((B, L, W), x.dtype),
        compiler_params=pltpu.CompilerParams(
            dimension_semantics=("parallel", "parallel"),
        ),
    )(x)

    return pl.pallas_call(
        _pe_body,
        grid=(B,),
        in_specs=[
            pl.BlockSpec(memory_space=pl.ANY),
            pl.BlockSpec((L, P), lambda b: (0, 0)),
        ],
        out_specs=pl.BlockSpec((1, L, P), lambda b: (b, 0, D // P)),
        out_shape=jax.ShapeDtypeStruct((B, L, W), x.dtype),
        input_output_aliases={0: 0},
        compiler_params=pltpu.CompilerParams(
            dimension_semantics=("arbitrary",),
        ),
    )(out1, x_pos)
